# baseline (speedup 1.0000x reference)
import functools

import jax
import jax.numpy as jnp
from jax import lax
from jax.experimental import pallas as pl
from jax.experimental.pallas import tpu as pltpu
from jax.experimental.pallas import tpu_sc as plsc


_NW = 32
_K = 80
_LANES = 16
_PAD = 17


def _pack_rows(y):
    w = y.shape[1] // 2
    u = lax.bitcast_convert_type(y.astype(jnp.bfloat16), jnp.uint16)
    lo = u[:, :w].astype(jnp.uint32)
    hi = u[:, w:].astype(jnp.uint32) << 16
    return lax.bitcast_convert_type(lo | hi, jnp.int32)


def _prep_body(x_ref, r_ref, xn_ref, rb_ref):
    xv = x_ref[...]
    n = jnp.sqrt(jnp.sum(xv * xv, axis=1, keepdims=True))
    xn_ref[...] = _pack_rows(xv / jnp.maximum(n, 1e-12))
    rb_ref[...] = _pack_rows(r_ref[...])


def _dot3(s_buf, o_buf, r_buf, e, j):
    sb, ob, rb = (
        plsc.bitcast(buf[e, pl.ds(j * _LANES, _LANES)], jnp.bfloat16)
        for buf in (s_buf, o_buf, r_buf))
    ta, tb = plsc.unpack(sb * ob * rb, format=plsc.PackFormat.INTERLEAVED,
                         preferred_element_type=jnp.float32)
    return ta + tb


def _score_body(n_w, e_w, xn_hbm, r_hbm, ei_hbm, et_hbm, out_hbm,
                src_v, dst_v, et_v, sA, oA, rA, sB, oB, rB, tmp_v, out_v,
                semA, semB):
    wid = lax.axis_index("s") * 2 + lax.axis_index("c")
    base = wid * e_w
    pltpu.sync_copy(ei_hbm.at[0, pl.ds(base, e_w)], src_v)
    pltpu.sync_copy(ei_hbm.at[1, pl.ds(base, e_w)], dst_v)
    pltpu.sync_copy(et_hbm.at[pl.ds(base, e_w)], et_v)
    n_chunks = e_w // _K
    lane17 = lax.iota(jnp.int32, _LANES) * _PAD
    bufs = {0: (sA, oA, rA, semA), 1: (sB, oB, rB, semB)}

    def copies(i, p):
        s_buf, o_buf, r_buf, sem = bufs[p]
        off = i * _K
        return (
            pltpu.make_async_copy(
                xn_hbm.at[src_v.at[pl.ds(off, _K)]], s_buf, sem),
            pltpu.make_async_copy(
                xn_hbm.at[dst_v.at[pl.ds(off, _K)]], o_buf, sem),
            pltpu.make_async_copy(
                r_hbm.at[et_v.at[pl.ds(off, _K)]], r_buf, sem),
        )

    def compute(i, p):
        s_buf, o_buf, r_buf, _ = bufs[p]
        off = i * _K

        @pl.loop(0, _K // _LANES, unroll=_K // _LANES)
        def _group(g):
            tbase = g * (_LANES * _PAD)
            for l in range(_LANES):
                e = g * _LANES + l
                acc = _dot3(s_buf, o_buf, r_buf, e, 0)
                for j in range(1, n_w // _LANES):
                    acc = acc + _dot3(s_buf, o_buf, r_buf, e, j)
                tmp_v[pl.ds(tbase + l * _PAD, _LANES)] = acc
            scores = plsc.load_gather(tmp_v, [lane17 + tbase])
            for k in range(1, _LANES):
                scores = scores + plsc.load_gather(tmp_v, [lane17 + (tbase + k)])
            out_v[pl.ds(off + g * _LANES, _LANES)] = scores

    for c in copies(0, 0):
        c.start()

    @pl.loop(0, (n_chunks + 1) // 2)
    def _pair(h):
        i0 = 2 * h
        i1 = i0 + 1

        @pl.when(i1 < n_chunks)
        def _():
            for c in copies(i1, 1):
                c.start()

        for c in copies(i0, 0):
            c.wait()
        compute(i0, 0)

        @pl.when(i1 < n_chunks)
        def _():
            @pl.when(i1 + 1 < n_chunks)
            def _():
                for c in copies(i1 + 1, 0):
                    c.start()

            for c in copies(i1, 1):
                c.wait()
            compute(i1, 1)

    pltpu.sync_copy(out_v, out_hbm.at[pl.ds(base, e_w)])


def kernel(x, edge_index, edge_type, R_diagonal):
    n_nodes, n_ch = x.shape
    n_rel = R_diagonal.shape[0]
    n_edges = edge_index.shape[1]
    n_w = n_ch // 2

    xn_w, r_w = pl.pallas_call(
        _prep_body,
        out_shape=(
            jax.ShapeDtypeStruct((n_nodes, n_w), jnp.int32),
            jax.ShapeDtypeStruct((n_rel, n_w), jnp.int32),
        ),
    )(x, R_diagonal)

    ei = edge_index.astype(jnp.int32)
    et = edge_type.astype(jnp.int32)

    e_w = n_edges // _NW
    mesh = plsc.VectorSubcoreMesh(core_axis_name="c", subcore_axis_name="s")
    score = pl.kernel(
        functools.partial(_score_body, n_w, e_w),
        out_type=jax.ShapeDtypeStruct((n_edges,), jnp.float32),
        mesh=mesh,
        compiler_params=pltpu.CompilerParams(needs_layout_passes=False,
                                             use_tc_tiling_on_sc=False),
        scratch_types=[
            pltpu.VMEM((e_w,), jnp.int32),
            pltpu.VMEM((e_w,), jnp.int32),
            pltpu.VMEM((e_w,), jnp.int32),
            pltpu.VMEM((_K, n_w), jnp.int32),
            pltpu.VMEM((_K, n_w), jnp.int32),
            pltpu.VMEM((_K, n_w), jnp.int32),
            pltpu.VMEM((_K, n_w), jnp.int32),
            pltpu.VMEM((_K, n_w), jnp.int32),
            pltpu.VMEM((_K, n_w), jnp.int32),
            pltpu.VMEM((_K // _LANES * _LANES * _PAD,), jnp.float32),
            pltpu.VMEM((e_w,), jnp.float32),
            pltpu.SemaphoreType.DMA,
            pltpu.SemaphoreType.DMA,
        ],
    )(xn_w, r_w, ei, et)
    return score


# speedup vs baseline: 1.1523x; 1.1523x over previous
import functools

import jax
import jax.numpy as jnp
from jax import lax
from jax.experimental import pallas as pl
from jax.experimental.pallas import tpu as pltpu
from jax.experimental.pallas import tpu_sc as plsc


_NW = 32
_K = 80
_LANES = 16


def _pack_rows(y):
    w = y.shape[1] // 2
    u = lax.bitcast_convert_type(y.astype(jnp.bfloat16), jnp.uint16)
    lo = u[:, :w].astype(jnp.uint32)
    hi = u[:, w:].astype(jnp.uint32) << 16
    return lax.bitcast_convert_type(lo | hi, jnp.int32)


def _prep_body(x_ref, r_ref, xn_ref, rb_ref):
    xv = x_ref[...]
    n = jnp.sqrt(jnp.sum(xv * xv, axis=1, keepdims=True))
    xn_ref[...] = _pack_rows(xv / jnp.maximum(n, 1e-12))
    rb_ref[...] = _pack_rows(r_ref[...])


def _dot3(s_buf, o_buf, r_buf, e, j):
    sb, ob, rb = (
        plsc.bitcast(buf[e, pl.ds(j * _LANES, _LANES)], jnp.bfloat16)
        for buf in (s_buf, o_buf, r_buf))
    ta, tb = plsc.unpack(sb * ob * rb, format=plsc.PackFormat.INTERLEAVED,
                         preferred_element_type=jnp.float32)
    return ta + tb


def _score_body(n_w, e_w, xn_hbm, r_hbm, ei_hbm, et_hbm, out_hbm,
                src_v, dst_v, et_v, sA, oA, rA, sB, oB, rB, out_v,
                semA, semB):
    wid = lax.axis_index("s") * 2 + lax.axis_index("c")
    base = wid * e_w
    pltpu.sync_copy(ei_hbm.at[0, pl.ds(base, e_w)], src_v)
    pltpu.sync_copy(ei_hbm.at[1, pl.ds(base, e_w)], dst_v)
    pltpu.sync_copy(et_hbm.at[pl.ds(base, e_w)], et_v)
    n_chunks = e_w // _K
    lane = lax.iota(jnp.int32, _LANES)
    lane_eq = [lane == l for l in range(_LANES)]
    bufs = {0: (sA, oA, rA, semA), 1: (sB, oB, rB, semB)}

    def copies(i, p):
        s_buf, o_buf, r_buf, sem = bufs[p]
        off = i * _K
        return (
            pltpu.make_async_copy(
                xn_hbm.at[src_v.at[pl.ds(off, _K)]], s_buf, sem),
            pltpu.make_async_copy(
                xn_hbm.at[dst_v.at[pl.ds(off, _K)]], o_buf, sem),
            pltpu.make_async_copy(
                r_hbm.at[et_v.at[pl.ds(off, _K)]], r_buf, sem),
        )

    def compute(i, p):
        s_buf, o_buf, r_buf, _ = bufs[p]
        off = i * _K

        @pl.loop(0, _K // _LANES)
        def _group(g):
            scores = jnp.zeros((_LANES,), jnp.float32)
            for l in range(_LANES):
                e = g * _LANES + l
                acc = _dot3(s_buf, o_buf, r_buf, e, 0)
                for j in range(1, n_w // _LANES):
                    acc = acc + _dot3(s_buf, o_buf, r_buf, e, j)
                scores = jnp.where(lane_eq[l], jnp.sum(acc), scores)
            out_v[pl.ds(off + g * _LANES, _LANES)] = scores

    for c in copies(0, 0):
        c.start()

    @pl.loop(0, (n_chunks + 1) // 2)
    def _pair(h):
        i0 = 2 * h
        i1 = i0 + 1

        @pl.when(i1 < n_chunks)
        def _():
            for c in copies(i1, 1):
                c.start()

        for c in copies(i0, 0):
            c.wait()
        compute(i0, 0)

        @pl.when(i1 < n_chunks)
        def _():
            @pl.when(i1 + 1 < n_chunks)
            def _():
                for c in copies(i1 + 1, 0):
                    c.start()

            for c in copies(i1, 1):
                c.wait()
            compute(i1, 1)

    pltpu.sync_copy(out_v, out_hbm.at[pl.ds(base, e_w)])


def kernel(x, edge_index, edge_type, R_diagonal):
    n_nodes, n_ch = x.shape
    n_rel = R_diagonal.shape[0]
    n_edges = edge_index.shape[1]
    n_w = n_ch // 2

    xn_w, r_w = pl.pallas_call(
        _prep_body,
        out_shape=(
            jax.ShapeDtypeStruct((n_nodes, n_w), jnp.int32),
            jax.ShapeDtypeStruct((n_rel, n_w), jnp.int32),
        ),
    )(x, R_diagonal)

    ei = edge_index.astype(jnp.int32)
    et = edge_type.astype(jnp.int32)

    e_w = n_edges // _NW
    mesh = plsc.VectorSubcoreMesh(core_axis_name="c", subcore_axis_name="s")
    score = pl.kernel(
        functools.partial(_score_body, n_w, e_w),
        out_type=jax.ShapeDtypeStruct((n_edges,), jnp.float32),
        mesh=mesh,
        compiler_params=pltpu.CompilerParams(needs_layout_passes=False,
                                             use_tc_tiling_on_sc=False),
        scratch_types=[
            pltpu.VMEM((e_w,), jnp.int32),
            pltpu.VMEM((e_w,), jnp.int32),
            pltpu.VMEM((e_w,), jnp.int32),
            pltpu.VMEM((_K, n_w), jnp.int32),
            pltpu.VMEM((_K, n_w), jnp.int32),
            pltpu.VMEM((_K, n_w), jnp.int32),
            pltpu.VMEM((_K, n_w), jnp.int32),
            pltpu.VMEM((_K, n_w), jnp.int32),
            pltpu.VMEM((_K, n_w), jnp.int32),
            pltpu.VMEM((e_w,), jnp.float32),
            pltpu.SemaphoreType.DMA,
            pltpu.SemaphoreType.DMA,
        ],
    )(xn_w, r_w, ei, et)
    return score


# speedup vs baseline: 1.3528x; 1.1740x over previous
import functools

import jax
import jax.numpy as jnp
from jax import lax
from jax.experimental import pallas as pl
from jax.experimental.pallas import tpu as pltpu
from jax.experimental.pallas import tpu_sc as plsc


_NW = 32
_K = 80
_LANES = 16
_PAD = 17


def _pack_rows(y):
    w = y.shape[1] // 2
    u = lax.bitcast_convert_type(y.astype(jnp.bfloat16), jnp.uint16)
    lo = u[:, :w].astype(jnp.uint32)
    hi = u[:, w:].astype(jnp.uint32) << 16
    return lax.bitcast_convert_type(lo | hi, jnp.int32)


def _prep_body(x_ref, r_ref, xn_ref, rb_ref):
    xv = x_ref[...]
    n = jnp.sqrt(jnp.sum(xv * xv, axis=1, keepdims=True))
    xn_ref[...] = _pack_rows(xv / jnp.maximum(n, 1e-12))
    rb_ref[...] = _pack_rows(r_ref[...])


def _dot3(s_buf, o_buf, r_buf, e, j):
    sb, ob, rb = (
        plsc.bitcast(buf[e, pl.ds(j * _LANES, _LANES)], jnp.bfloat16)
        for buf in (s_buf, o_buf, r_buf))
    ta, tb = plsc.unpack(sb * ob * rb, format=plsc.PackFormat.INTERLEAVED,
                         preferred_element_type=jnp.float32)
    return ta + tb


def _score_body(n_w, e_w, xn_hbm, r_hbm, ei_hbm, et_hbm, out_hbm,
                src_v, dst_v, et_v, sA, oA, rA, sB, oB, rB, tmp_v, out_v,
                semA, semB):
    wid = lax.axis_index("s") * 2 + lax.axis_index("c")
    base = wid * e_w
    pltpu.sync_copy(ei_hbm.at[0, pl.ds(base, e_w)], src_v)
    pltpu.sync_copy(ei_hbm.at[1, pl.ds(base, e_w)], dst_v)
    pltpu.sync_copy(et_hbm.at[pl.ds(base, e_w)], et_v)
    n_chunks = e_w // _K
    lane17 = lax.iota(jnp.int32, _LANES) * _PAD
    bufs = {0: (sA, oA, rA, semA), 1: (sB, oB, rB, semB)}

    def copies(i, p):
        s_buf, o_buf, r_buf, sem = bufs[p]
        off = i * _K
        return (
            pltpu.make_async_copy(
                xn_hbm.at[src_v.at[pl.ds(off, _K)]], s_buf, sem),
            pltpu.make_async_copy(
                xn_hbm.at[dst_v.at[pl.ds(off, _K)]], o_buf, sem),
            pltpu.make_async_copy(
                r_hbm.at[et_v.at[pl.ds(off, _K)]], r_buf, sem),
        )

    def compute(i, p):
        s_buf, o_buf, r_buf, _ = bufs[p]
        off = i * _K

        @pl.loop(0, _K // _LANES)
        def _group(g):
            for l in range(_LANES):
                e = g * _LANES + l
                acc = _dot3(s_buf, o_buf, r_buf, e, 0)
                for j in range(1, n_w // _LANES):
                    acc = acc + _dot3(s_buf, o_buf, r_buf, e, j)
                tmp_v[pl.ds(l * _PAD, _LANES)] = acc
            scores = plsc.load_gather(tmp_v, [lane17])
            for k in range(1, _LANES):
                scores = scores + plsc.load_gather(tmp_v, [lane17 + k])
            out_v[pl.ds(off + g * _LANES, _LANES)] = scores

    for c in copies(0, 0):
        c.start()

    @pl.loop(0, (n_chunks + 1) // 2)
    def _pair(h):
        i0 = 2 * h
        i1 = i0 + 1

        @pl.when(i1 < n_chunks)
        def _():
            for c in copies(i1, 1):
                c.start()

        for c in copies(i0, 0):
            c.wait()
        compute(i0, 0)

        @pl.when(i1 < n_chunks)
        def _():
            @pl.when(i1 + 1 < n_chunks)
            def _():
                for c in copies(i1 + 1, 0):
                    c.start()

            for c in copies(i1, 1):
                c.wait()
            compute(i1, 1)

    pltpu.sync_copy(out_v, out_hbm.at[pl.ds(base, e_w)])


def kernel(x, edge_index, edge_type, R_diagonal):
    n_nodes, n_ch = x.shape
    n_rel = R_diagonal.shape[0]
    n_edges = edge_index.shape[1]
    n_w = n_ch // 2

    xn_w, r_w = pl.pallas_call(
        _prep_body,
        out_shape=(
            jax.ShapeDtypeStruct((n_nodes, n_w), jnp.int32),
            jax.ShapeDtypeStruct((n_rel, n_w), jnp.int32),
        ),
    )(x, R_diagonal)

    ei = edge_index.astype(jnp.int32)
    et = edge_type.astype(jnp.int32)

    e_w = n_edges // _NW
    mesh = plsc.VectorSubcoreMesh(core_axis_name="c", subcore_axis_name="s")
    score = pl.kernel(
        functools.partial(_score_body, n_w, e_w),
        out_type=jax.ShapeDtypeStruct((n_edges,), jnp.float32),
        mesh=mesh,
        compiler_params=pltpu.CompilerParams(needs_layout_passes=False,
                                             use_tc_tiling_on_sc=False),
        scratch_types=[
            pltpu.VMEM((e_w,), jnp.int32),
            pltpu.VMEM((e_w,), jnp.int32),
            pltpu.VMEM((e_w,), jnp.int32),
            pltpu.VMEM((_K, n_w), jnp.int32),
            pltpu.VMEM((_K, n_w), jnp.int32),
            pltpu.VMEM((_K, n_w), jnp.int32),
            pltpu.VMEM((_K, n_w), jnp.int32),
            pltpu.VMEM((_K, n_w), jnp.int32),
            pltpu.VMEM((_K, n_w), jnp.int32),
            pltpu.VMEM((_LANES * _PAD,), jnp.float32),
            pltpu.VMEM((e_w,), jnp.float32),
            pltpu.SemaphoreType.DMA,
            pltpu.SemaphoreType.DMA,
        ],
    )(xn_w, r_w, ei, et)
    return score
